# baseline (speedup 1.0000x reference)
import dataclasses
import functools

import jax
import jax.numpy as jnp
from jax import lax
from jax.experimental import pallas as pl
from jax.experimental.pallas import tpu as pltpu
from jax.experimental.pallas import tpu_sc as plsc

NN = 10000
EE = 320000
DD = 128
CC = 40
CP = 64

NC = 2
NS = 16
NW = NC * NS
LL = 16

BB = 128
KK = 80
GG = 8
EW = KK * BB
EP = NW * EW
NACC = 10240
NH = 10016
ZB = NACC // BB // NS
RO = NACC // NS

_MESH = dict(core_axis_name="c", subcore_axis_name="s")

_SC_PARAMS = pltpu.CompilerParams()
if "needs_layout_passes" in pltpu.CompilerParams.__dataclass_fields__:
    _SC_PARAMS = dataclasses.replace(_SC_PARAMS, needs_layout_passes=False)
_SC_PARAMS_NARROW = dataclasses.replace(_SC_PARAMS, use_tc_tiling_on_sc=False)


def _deg_sc(dst_flat):
    @functools.partial(
        pl.kernel,
        out_type=jax.ShapeDtypeStruct((NW, NH), jnp.float32),
        mesh=plsc.VectorSubcoreMesh(**_MESH),
        scratch_types=[
            pltpu.VMEM((EW,), jnp.int32),
            pltpu.VMEM((NH,), jnp.float32),
        ],
        compiler_params=_SC_PARAMS,
    )
    def k(dst_hbm, out_hbm, idx_v, hist_v):
        s = lax.axis_index("s")
        c = lax.axis_index("c")
        w = s * NC + c
        pltpu.sync_copy(dst_hbm.at[w], idx_v)
        zeros = jnp.zeros((LL,), jnp.float32)
        ones = jnp.ones((LL,), jnp.float32)

        @pl.loop(0, NH, step=LL)
        def _(i):
            hist_v[pl.ds(i, LL)] = zeros

        @pl.loop(0, EW, step=LL)
        def _(e):
            idx = idx_v[pl.ds(e, LL)]
            plsc.addupdate_scatter(hist_v, [idx], ones)

        pltpu.sync_copy(hist_v, out_hbm.at[w])

    return k(dst_flat)


def _agg_sc(hp, srcp, dstp, d):
    @functools.partial(
        pl.kernel,
        out_type=jax.ShapeDtypeStruct((NC, NACC, d), jnp.float32),
        mesh=plsc.VectorSubcoreMesh(**_MESH),
        scratch_types=[
            pltpu.VMEM((GG, BB), jnp.int32),
            pltpu.VMEM((GG, BB), jnp.int32),
            pltpu.VMEM((BB, d), jnp.float32),
            pltpu.VMEM_SHARED((NACC, d), jnp.float32),
        ],
        compiler_params=_SC_PARAMS if d == DD else _SC_PARAMS_NARROW,
    )
    def k(hp_hbm, src_hbm, dst_hbm, out_hbm, src_v, dst_v, a0, acc_sh):
        s = lax.axis_index("s")
        c = lax.axis_index("c")
        w = s * NC + c

        zeros = jnp.zeros((LL,), jnp.float32)

        @pl.loop(0, BB)
        def _(r):
            @pl.loop(0, d, step=LL)
            def _(cc):
                a0[r, pl.ds(cc, LL)] = zeros

        @pl.loop(0, ZB)
        def _(t):
            pltpu.sync_copy(a0, acc_sh.at[pl.ds((s * ZB + t) * BB, BB)])

        plsc.subcore_barrier()

        @pl.loop(0, KK // GG)
        def _(g):
            pltpu.sync_copy(src_hbm.at[w, pl.ds(g * GG, GG)], src_v)
            pltpu.sync_copy(dst_hbm.at[w, pl.ds(g * GG, GG)], dst_v)
            for j in range(GG):
                pltpu.sync_copy(hp_hbm.at[src_v.at[j]], a0)
                pltpu.sync_copy(a0, acc_sh.at[dst_v.at[j]], add=True)

        plsc.subcore_barrier()
        pltpu.sync_copy(acc_sh.at[pl.ds(s * RO, RO)],
                        out_hbm.at[c, pl.ds(s * RO, RO)])

    return k(hp, srcp, dstp)


def _mm1_tc(x, w1):
    def body(x_ref, w_ref, o_ref):
        o_ref[...] = jnp.dot(x_ref[...], w_ref[...],
                             preferred_element_type=jnp.float32)

    return pl.pallas_call(
        body,
        grid=(10,),
        in_specs=[
            pl.BlockSpec((1000, DD), lambda i: (i, 0)),
            pl.BlockSpec((DD, DD), lambda i: (0, 0)),
        ],
        out_specs=pl.BlockSpec((1000, DD), lambda i: (i, 0)),
        out_shape=jax.ShapeDtypeStruct((NN, DD), jnp.float32),
    )(x, w1)


def _prep_tc(hist, xw1):
    def body(h_ref, xw_ref, dinv_ref, h1p_ref):
        ones32 = jnp.ones((NW, 1), jnp.float32)
        deg = lax.dot_general(h_ref[...], ones32, (((0,), (0,)), ((), ())),
                              precision=lax.Precision.HIGHEST,
                              preferred_element_type=jnp.float32)
        dinv = lax.rsqrt(deg[:NN] + 1.0)
        db = jnp.broadcast_to(dinv, (NN, DD))
        dinv_ref[...] = db
        h1p_ref[...] = db * xw_ref[...]

    return pl.pallas_call(
        body,
        out_shape=[
            jax.ShapeDtypeStruct((NN, DD), jnp.float32),
            jax.ShapeDtypeStruct((NN, DD), jnp.float32),
        ],
    )(hist, xw1)


def _post1_tc(p1, h1p, dinvb, w2p):
    def body(p_ref, h_ref, dv_ref, w_ref, o_ref):
        agg = p_ref[0] + p_ref[1] + h_ref[...]
        h1 = jnp.maximum(dv_ref[...] * agg, 0.0)
        mm = jnp.dot(h1, w_ref[...], preferred_element_type=jnp.float32)
        o_ref[...] = dv_ref[:, :CP] * mm

    return pl.pallas_call(
        body,
        grid=(10,),
        in_specs=[
            pl.BlockSpec((NC, 1000, DD), lambda i: (0, i, 0)),
            pl.BlockSpec((1000, DD), lambda i: (i, 0)),
            pl.BlockSpec((1000, DD), lambda i: (i, 0)),
            pl.BlockSpec((DD, CP), lambda i: (0, 0)),
        ],
        out_specs=pl.BlockSpec((1000, CP), lambda i: (i, 0)),
        out_shape=jax.ShapeDtypeStruct((NN, CP), jnp.float32),
    )(p1, h1p, dinvb, w2p)


def _post2_tc(p2, h2p, dinvb):
    def body(p_ref, h_ref, dv_ref, o_ref):
        agg = p_ref[0] + p_ref[1] + h_ref[...]
        o_ref[...] = jnp.maximum(dv_ref[:, :CC] * agg[:, :CC], 0.0)

    return pl.pallas_call(
        body,
        grid=(10,),
        in_specs=[
            pl.BlockSpec((NC, 1000, CP), lambda i: (0, i, 0)),
            pl.BlockSpec((1000, CP), lambda i: (i, 0)),
            pl.BlockSpec((1000, DD), lambda i: (i, 0)),
        ],
        out_specs=pl.BlockSpec((1000, CC), lambda i: (i, 0)),
        out_shape=jax.ShapeDtypeStruct((NN, CC), jnp.float32),
    )(p2, h2p, dinvb)


def kernel(x, edge_index, W1, W2):
    pad = EP - EE
    srcp = jnp.concatenate(
        [edge_index[0], jnp.zeros((pad,), jnp.int32)]).reshape(NW, KK, BB)
    dstp = jnp.concatenate(
        [edge_index[1], jnp.full((pad,), NN, jnp.int32)]).reshape(NW, KK, BB)
    w2p = jnp.pad(W2, ((0, 0), (0, CP - CC)))

    hist = _deg_sc(dstp.reshape(NW, EW))
    xw1 = _mm1_tc(x, W1)
    dinvb, h1p = _prep_tc(hist, xw1)
    p1 = _agg_sc(h1p, srcp, dstp, DD)
    h2p = _post1_tc(p1, h1p, dinvb, w2p)
    p2 = _agg_sc(h2p, srcp, dstp, CP)
    return _post2_tc(p2, h2p, dinvb)


# speedup vs baseline: 1.0067x; 1.0067x over previous
import dataclasses
import functools

import jax
import jax.numpy as jnp
from jax import lax
from jax.experimental import pallas as pl
from jax.experimental.pallas import tpu as pltpu
from jax.experimental.pallas import tpu_sc as plsc

NN = 10000
EE = 320000
DD = 128
CC = 40
CP = 64

NC = 2
NS = 16
NW = NC * NS
LL = 16

BB = 128
KK = 80
GG = 8
EW = KK * BB
EP = NW * EW
NACC = 10240
NH = 10240
ZB = NACC // BB // NS
RO = NACC // NS

_MESH = dict(core_axis_name="c", subcore_axis_name="s")

_SC_PARAMS = pltpu.CompilerParams()
if "needs_layout_passes" in pltpu.CompilerParams.__dataclass_fields__:
    _SC_PARAMS = dataclasses.replace(_SC_PARAMS, needs_layout_passes=False)
_SC_PARAMS_NARROW = dataclasses.replace(_SC_PARAMS, use_tc_tiling_on_sc=False)


def _deg_sc(dst_flat):
    @functools.partial(
        pl.kernel,
        out_type=jax.ShapeDtypeStruct((NW, NH), jnp.float32),
        mesh=plsc.VectorSubcoreMesh(**_MESH),
        scratch_types=[
            pltpu.VMEM((EW,), jnp.int32),
            pltpu.VMEM((NH,), jnp.float32),
        ],
        compiler_params=_SC_PARAMS,
    )
    def k(dst_hbm, out_hbm, idx_v, hist_v):
        s = lax.axis_index("s")
        c = lax.axis_index("c")
        w = s * NC + c
        pltpu.sync_copy(dst_hbm.at[w], idx_v)
        zeros = jnp.zeros((LL,), jnp.float32)
        ones = jnp.ones((LL,), jnp.float32)

        @pl.loop(0, NH, step=LL)
        def _(i):
            hist_v[pl.ds(i, LL)] = zeros

        @pl.loop(0, EW, step=LL)
        def _(e):
            idx = idx_v[pl.ds(e, LL)]
            plsc.addupdate_scatter(hist_v, [idx], ones)

        pltpu.sync_copy(hist_v, out_hbm.at[w])

    return k(dst_flat)


def _agg_sc(hp, srcp, dstp, d):
    @functools.partial(
        pl.kernel,
        out_type=jax.ShapeDtypeStruct((NC, NACC, d), jnp.float32),
        mesh=plsc.VectorSubcoreMesh(**_MESH),
        scratch_types=[
            pltpu.VMEM((GG, BB), jnp.int32),
            pltpu.VMEM((GG, BB), jnp.int32),
            pltpu.VMEM((BB, d), jnp.float32),
            pltpu.VMEM_SHARED((NACC, d), jnp.float32),
        ],
        compiler_params=_SC_PARAMS if d == DD else _SC_PARAMS_NARROW,
    )
    def k(hp_hbm, src_hbm, dst_hbm, out_hbm, src_v, dst_v, a0, acc_sh):
        s = lax.axis_index("s")
        c = lax.axis_index("c")
        w = s * NC + c

        zeros = jnp.zeros((LL,), jnp.float32)

        @pl.loop(0, BB)
        def _(r):
            @pl.loop(0, d, step=LL)
            def _(cc):
                a0[r, pl.ds(cc, LL)] = zeros

        @pl.loop(0, ZB)
        def _(t):
            pltpu.sync_copy(a0, acc_sh.at[pl.ds((s * ZB + t) * BB, BB)])

        plsc.subcore_barrier()

        @pl.loop(0, KK // GG)
        def _(g):
            pltpu.sync_copy(src_hbm.at[w, pl.ds(g * GG, GG)], src_v)
            pltpu.sync_copy(dst_hbm.at[w, pl.ds(g * GG, GG)], dst_v)
            for j in range(GG):
                pltpu.sync_copy(hp_hbm.at[src_v.at[j]], a0)
                pltpu.sync_copy(a0, acc_sh.at[dst_v.at[j]], add=True)

        plsc.subcore_barrier()
        pltpu.sync_copy(acc_sh.at[pl.ds(s * RO, RO)],
                        out_hbm.at[c, pl.ds(s * RO, RO)])

    return k(hp, srcp, dstp)


def _mm1_tc(x, w1):
    def body(x_ref, w_ref, o_ref):
        o_ref[...] = jnp.dot(x_ref[...], w_ref[...],
                             preferred_element_type=jnp.float32)

    return pl.pallas_call(
        body,
        grid=(10,),
        in_specs=[
            pl.BlockSpec((1000, DD), lambda i: (i, 0)),
            pl.BlockSpec((DD, DD), lambda i: (0, 0)),
        ],
        out_specs=pl.BlockSpec((1000, DD), lambda i: (i, 0)),
        out_shape=jax.ShapeDtypeStruct((NN, DD), jnp.float32),
    )(x, w1)


def _prep_tc(hist, xw1):
    def body(h_ref, xw_ref, dinv_ref, h1p_ref):
        ones32 = jnp.ones((NW, 1), jnp.float32)
        deg = lax.dot_general(h_ref[...], ones32, (((0,), (0,)), ((), ())),
                              precision=lax.Precision.HIGHEST,
                              preferred_element_type=jnp.float32)
        dinv = lax.rsqrt(deg[:NN] + 1.0)
        db = jnp.broadcast_to(dinv, (NN, DD))
        dinv_ref[...] = db
        h1p_ref[...] = db * xw_ref[...]

    return pl.pallas_call(
        body,
        out_shape=[
            jax.ShapeDtypeStruct((NN, DD), jnp.float32),
            jax.ShapeDtypeStruct((NN, DD), jnp.float32),
        ],
    )(hist, xw1)


def _post1_tc(p1, h1p, dinvb, w2p):
    def body(p_ref, h_ref, dv_ref, w_ref, o_ref):
        agg = p_ref[0] + p_ref[1] + h_ref[...]
        h1 = jnp.maximum(dv_ref[...] * agg, 0.0)
        mm = jnp.dot(h1, w_ref[...], preferred_element_type=jnp.float32)
        o_ref[...] = dv_ref[:, :CP] * mm

    return pl.pallas_call(
        body,
        grid=(10,),
        in_specs=[
            pl.BlockSpec((NC, 1000, DD), lambda i: (0, i, 0)),
            pl.BlockSpec((1000, DD), lambda i: (i, 0)),
            pl.BlockSpec((1000, DD), lambda i: (i, 0)),
            pl.BlockSpec((DD, CP), lambda i: (0, 0)),
        ],
        out_specs=pl.BlockSpec((1000, CP), lambda i: (i, 0)),
        out_shape=jax.ShapeDtypeStruct((NN, CP), jnp.float32),
    )(p1, h1p, dinvb, w2p)


def _post2_tc(p2, h2p, dinvb):
    def body(p_ref, h_ref, dv_ref, o_ref):
        agg = p_ref[0] + p_ref[1] + h_ref[...]
        o_ref[...] = jnp.maximum(dv_ref[:, :CC] * agg[:, :CC], 0.0)

    return pl.pallas_call(
        body,
        grid=(10,),
        in_specs=[
            pl.BlockSpec((NC, 1000, CP), lambda i: (0, i, 0)),
            pl.BlockSpec((1000, CP), lambda i: (i, 0)),
            pl.BlockSpec((1000, DD), lambda i: (i, 0)),
        ],
        out_specs=pl.BlockSpec((1000, CC), lambda i: (i, 0)),
        out_shape=jax.ShapeDtypeStruct((NN, CC), jnp.float32),
    )(p2, h2p, dinvb)


def kernel(x, edge_index, W1, W2):
    pad = EP - EE
    trash = NN + jnp.arange(pad, dtype=jnp.int32) % (NACC - NN)
    srcp = jnp.concatenate(
        [edge_index[0], jnp.zeros((pad,), jnp.int32)]).reshape(NW, KK, BB)
    dstp = jnp.concatenate([edge_index[1], trash]).reshape(NW, KK, BB)
    w2p = jnp.pad(W2, ((0, 0), (0, CP - CC)))

    hist = _deg_sc(dstp.reshape(NW, EW))
    xw1 = _mm1_tc(x, W1)
    dinvb, h1p = _prep_tc(hist, xw1)
    p1 = _agg_sc(h1p, srcp, dstp, DD)
    h2p = _post1_tc(p1, h1p, dinvb, w2p)
    p2 = _agg_sc(h2p, srcp, dstp, CP)
    return _post2_tc(p2, h2p, dinvb)


# speedup vs baseline: 1.6728x; 1.6617x over previous
import dataclasses
import functools

import jax
import jax.numpy as jnp
from jax import lax
from jax.experimental import pallas as pl
from jax.experimental.pallas import tpu as pltpu
from jax.experimental.pallas import tpu_sc as plsc

NN = 10000
EE = 320000
DD = 128
CC = 40
CP = 64

NC = 2
NS = 16
NW = NC * NS
LL = 16

BB = 128
KK = 79
EW = KK * BB
EP = NW * EW
NACC = 10240
NH = 10016
ZB = NACC // BB // NS
RO = NACC // NS

_MESH = dict(core_axis_name="c", subcore_axis_name="s")

_SC_PARAMS = pltpu.CompilerParams()
if "needs_layout_passes" in pltpu.CompilerParams.__dataclass_fields__:
    _SC_PARAMS = dataclasses.replace(_SC_PARAMS, needs_layout_passes=False)
_SC_PARAMS_NARROW = dataclasses.replace(_SC_PARAMS, use_tc_tiling_on_sc=False)


def _deg_sc(dst_flat):
    @functools.partial(
        pl.kernel,
        out_type=jax.ShapeDtypeStruct((NW, NH), jnp.float32),
        mesh=plsc.VectorSubcoreMesh(**_MESH),
        scratch_types=[
            pltpu.VMEM((EW,), jnp.int32),
            pltpu.VMEM((NH,), jnp.float32),
        ],
        compiler_params=_SC_PARAMS,
    )
    def k(dst_hbm, out_hbm, idx_v, hist_v):
        s = lax.axis_index("s")
        c = lax.axis_index("c")
        w = s * NC + c
        pltpu.sync_copy(dst_hbm.at[w], idx_v)
        zeros = jnp.zeros((LL,), jnp.float32)
        ones = jnp.ones((LL,), jnp.float32)

        @pl.loop(0, NH, step=LL)
        def _(i):
            hist_v[pl.ds(i, LL)] = zeros

        @pl.loop(0, EW, step=LL)
        def _(e):
            idx = idx_v[pl.ds(e, LL)]
            plsc.addupdate_scatter(hist_v, [idx], ones)

        pltpu.sync_copy(hist_v, out_hbm.at[w])

    return k(dst_flat)


def _agg_sc(hp, srcp, dstp, d):
    @functools.partial(
        pl.kernel,
        out_type=jax.ShapeDtypeStruct((NC, NACC, d), jnp.float32),
        mesh=plsc.VectorSubcoreMesh(**_MESH),
        scratch_types=[
            pltpu.VMEM((KK, BB), jnp.int32),
            pltpu.VMEM((KK, BB), jnp.int32),
            pltpu.VMEM((BB, d), jnp.float32),
            pltpu.VMEM_SHARED((NACC, d), jnp.float32),
        ],
        compiler_params=_SC_PARAMS if d == DD else _SC_PARAMS_NARROW,
    )
    def k(hp_hbm, src_hbm, dst_hbm, out_hbm, src_v, dst_v, a0, acc_sh):
        s = lax.axis_index("s")
        c = lax.axis_index("c")
        w = s * NC + c

        zeros = jnp.zeros((LL,), jnp.float32)

        @pl.loop(0, BB)
        def _(r):
            @pl.loop(0, d, step=LL)
            def _(cc):
                a0[r, pl.ds(cc, LL)] = zeros

        @pl.loop(0, ZB)
        def _(t):
            pltpu.sync_copy(a0, acc_sh.at[pl.ds((s * ZB + t) * BB, BB)])

        plsc.subcore_barrier()

        pltpu.sync_copy(src_hbm.at[w], src_v)
        pltpu.sync_copy(dst_hbm.at[w], dst_v)
        for j in range(KK):
            pltpu.sync_copy(hp_hbm.at[src_v.at[j]], a0)
            pltpu.sync_copy(a0, acc_sh.at[dst_v.at[j]], add=True)

        plsc.subcore_barrier()
        pltpu.sync_copy(acc_sh.at[pl.ds(s * RO, RO)],
                        out_hbm.at[c, pl.ds(s * RO, RO)])

    return k(hp, srcp, dstp)


def _mm1_tc(x, w1):
    def body(x_ref, w_ref, o_ref):
        o_ref[...] = jnp.dot(x_ref[...], w_ref[...],
                             preferred_element_type=jnp.float32)

    return pl.pallas_call(
        body,
        grid=(10,),
        in_specs=[
            pl.BlockSpec((1000, DD), lambda i: (i, 0)),
            pl.BlockSpec((DD, DD), lambda i: (0, 0)),
        ],
        out_specs=pl.BlockSpec((1000, DD), lambda i: (i, 0)),
        out_shape=jax.ShapeDtypeStruct((NN, DD), jnp.float32),
    )(x, w1)


def _prep_tc(hist, xw1):
    def body(h_ref, xw_ref, dinv_ref, h1p_ref):
        ones32 = jnp.ones((NW, 1), jnp.float32)
        deg = lax.dot_general(h_ref[...], ones32, (((0,), (0,)), ((), ())),
                              precision=lax.Precision.HIGHEST,
                              preferred_element_type=jnp.float32)
        dinv = lax.rsqrt(deg[:NN] + 1.0)
        db = jnp.broadcast_to(dinv, (NN, DD))
        dinv_ref[...] = db
        h1p_ref[...] = db * xw_ref[...]

    return pl.pallas_call(
        body,
        out_shape=[
            jax.ShapeDtypeStruct((NN, DD), jnp.float32),
            jax.ShapeDtypeStruct((NN, DD), jnp.float32),
        ],
    )(hist, xw1)


def _post1_tc(p1, h1p, dinvb, w2p):
    def body(p_ref, h_ref, dv_ref, w_ref, o_ref):
        agg = p_ref[0] + p_ref[1] + h_ref[...]
        h1 = jnp.maximum(dv_ref[...] * agg, 0.0)
        mm = jnp.dot(h1, w_ref[...], preferred_element_type=jnp.float32)
        o_ref[...] = dv_ref[:, :CP] * mm

    return pl.pallas_call(
        body,
        grid=(10,),
        in_specs=[
            pl.BlockSpec((NC, 1000, DD), lambda i: (0, i, 0)),
            pl.BlockSpec((1000, DD), lambda i: (i, 0)),
            pl.BlockSpec((1000, DD), lambda i: (i, 0)),
            pl.BlockSpec((DD, CP), lambda i: (0, 0)),
        ],
        out_specs=pl.BlockSpec((1000, CP), lambda i: (i, 0)),
        out_shape=jax.ShapeDtypeStruct((NN, CP), jnp.float32),
    )(p1, h1p, dinvb, w2p)


def _post2_tc(p2, h2p, dinvb):
    def body(p_ref, h_ref, dv_ref, o_ref):
        agg = p_ref[0] + p_ref[1] + h_ref[...]
        o_ref[...] = jnp.maximum(dv_ref[:, :CC] * agg[:, :CC], 0.0)

    return pl.pallas_call(
        body,
        grid=(10,),
        in_specs=[
            pl.BlockSpec((NC, 1000, CP), lambda i: (0, i, 0)),
            pl.BlockSpec((1000, CP), lambda i: (i, 0)),
            pl.BlockSpec((1000, DD), lambda i: (i, 0)),
        ],
        out_specs=pl.BlockSpec((1000, CC), lambda i: (i, 0)),
        out_shape=jax.ShapeDtypeStruct((NN, CC), jnp.float32),
    )(p2, h2p, dinvb)


def kernel(x, edge_index, W1, W2):
    pad = EP - EE
    trash = jnp.full((pad,), NN, jnp.int32)
    srcp = jnp.concatenate(
        [edge_index[0], jnp.zeros((pad,), jnp.int32)]).reshape(NW, KK, BB)
    dstp = jnp.concatenate([edge_index[1], trash]).reshape(NW, KK, BB)
    w2p = jnp.pad(W2, ((0, 0), (0, CP - CC)))

    hist = _deg_sc(dstp.reshape(NW, EW))
    xw1 = _mm1_tc(x, W1)
    dinvb, h1p = _prep_tc(hist, xw1)
    p1 = _agg_sc(h1p, srcp, dstp, DD)
    h2p = _post1_tc(p1, h1p, dinvb, w2p)
    p2 = _agg_sc(h2p, srcp, dstp, CP)
    return _post2_tc(p2, h2p, dinvb)
